# baseline (speedup 1.0000x reference)
import dataclasses
import functools

import jax
import jax.numpy as jnp
from jax import lax
from jax.experimental import pallas as pl
from jax.experimental.pallas import tpu as pltpu
from jax.experimental.pallas import tpu_sc as plsc

N = 10000
E = 320000
D = 128
L = 16
NC = 2
NS = 16
NW = NC * NS
EPW = E // NW
G = 80
NIT = EPW // G
NPAD = 10240
RPS = NPAD // NS
NP8 = NPAD // 8
RPS8 = NP8 // NS

_mesh = plsc.VectorSubcoreMesh(core_axis_name="c", subcore_axis_name="s")

_cp = pltpu.CompilerParams()
if "needs_layout_passes" in pltpu.CompilerParams.__dataclass_fields__:
    _cp = dataclasses.replace(_cp, needs_layout_passes=False)


def _project_body(h_ref, wfc_ref, wfcr_ref, am_ref, z_ref, zr_ref, s_ref):
    hb = h_ref[...]
    dn = (((1,), (1,)), ((), ()))
    z = lax.dot_general(hb, wfc_ref[...], dn,
                        preferred_element_type=jnp.float32,
                        precision=lax.Precision.HIGHEST)
    zr = lax.dot_general(hb, wfcr_ref[...], dn,
                         preferred_element_type=jnp.float32,
                         precision=lax.Precision.HIGHEST)
    am = am_ref[...]
    dn2 = (((1,), (0,)), ((), ()))
    sz = lax.dot_general(z, am, dn2, preferred_element_type=jnp.float32,
                         precision=lax.Precision.HIGHEST)
    szr = lax.dot_general(zr, am, dn2, preferred_element_type=jnp.float32,
                          precision=lax.Precision.HIGHEST)
    z_ref[...] = z
    zr_ref[...] = zr
    s_ref[...] = jnp.concatenate([sz, szr], axis=1)


def _project(h, W_fc, W_fcr, am):
    R = 1000
    return pl.pallas_call(
        _project_body,
        grid=(N // R,),
        in_specs=[
            pl.BlockSpec((R, D), lambda i: (i, 0)),
            pl.BlockSpec((D, D), lambda i: (0, 0)),
            pl.BlockSpec((D, D), lambda i: (0, 0)),
            pl.BlockSpec((D, 2), lambda i: (0, 0)),
        ],
        out_specs=[
            pl.BlockSpec((R, D), lambda i: (i, 0)),
            pl.BlockSpec((R, D), lambda i: (i, 0)),
            pl.BlockSpec((R, 4), lambda i: (i, 0)),
        ],
        out_shape=[
            jax.ShapeDtypeStruct((N, D), jnp.float32),
            jax.ShapeDtypeStruct((N, D), jnp.float32),
            jax.ShapeDtypeStruct((N, 4), jnp.float32),
        ],
    )(h, W_fc, W_fcr, am)


def _lrelu(x):
    return jnp.where(x >= 0.0, x, 0.01 * x)


@functools.partial(
    pl.kernel,
    mesh=_mesh,
    out_type=[
        jax.ShapeDtypeStruct((E,), jnp.float32),
        jax.ShapeDtypeStruct((E,), jnp.float32),
        jax.ShapeDtypeStruct((NW, L), jnp.float32),
        jax.ShapeDtypeStruct((NC, NP8, D), jnp.float32),
    ],
    scratch_types=[
        pltpu.VMEM((N,), jnp.float32),
        pltpu.VMEM((N,), jnp.float32),
        pltpu.VMEM((N,), jnp.float32),
        pltpu.VMEM((N,), jnp.float32),
        pltpu.VMEM((EPW,), jnp.int32),
        pltpu.VMEM((EPW,), jnp.int32),
        pltpu.VMEM((EPW,), jnp.float32),
        pltpu.VMEM((EPW,), jnp.float32),
        pltpu.VMEM((EPW,), jnp.float32),
        pltpu.VMEM((L,), jnp.float32),
        pltpu.VMEM((8, D), jnp.float32),
        pltpu.VMEM((NS * 8, D), jnp.float32),
        pltpu.VMEM((G, D), jnp.float32),
        pltpu.VMEM((2, G), jnp.int32),
        pltpu.VMEM_SHARED((NP8, D), jnp.float32),
        pltpu.VMEM_SHARED((NS * 8, D), jnp.float32),
    ],
    compiler_params=_cp,
)
def _edge_scores(s1_hbm, s2_hbm, sr1_hbm, sr2_hbm, src_hbm, dst_hbm,
                 d0_hbm, d1_hbm, c0_hbm, c1_hbm, mx_hbm, pden_hbm,
                 s1v, s2v, sr1v, sr2v, srcv, dstv, d0v, d1v, efv, mxv,
                 mxw, mxallv, denb, idxb, sden, mxsh):
    cid = lax.axis_index("c")
    sid = lax.axis_index("s")
    wid = cid * NS + sid
    base = wid * EPW
    pltpu.sync_copy(s1_hbm, s1v)
    pltpu.sync_copy(s2_hbm, s2v)
    pltpu.sync_copy(sr1_hbm, sr1v)
    pltpu.sync_copy(sr2_hbm, sr2v)
    pltpu.sync_copy(src_hbm.at[pl.ds(base, EPW)], srcv)
    pltpu.sync_copy(dst_hbm.at[pl.ds(base, EPW)], dstv)
    pltpu.sync_copy(d0_hbm.at[pl.ds(base, EPW)], d0v)
    pltpu.sync_copy(d1_hbm.at[pl.ds(base, EPW)], d1v)
    mxv[...] = jnp.full((L,), -3e38, jnp.float32)

    z16 = jnp.zeros((L,), jnp.float32)

    @pl.loop(0, G)
    def _(r):
        for j in range(D // L):
            denb[r, pl.ds(j * L, L)] = z16

    pltpu.sync_copy(denb, sden.at[pl.ds(sid * RPS8, RPS8)])

    @pl.loop(0, EPW, step=L)
    def _(g):
        sl = pl.ds(g, L)
        i16 = srcv[sl]
        j16 = dstv[sl]
        a1 = plsc.load_gather(s1v, [i16])
        a2 = plsc.load_gather(s2v, [j16])
        b1 = plsc.load_gather(sr1v, [i16])
        b2 = plsc.load_gather(sr2v, [j16])
        ef16 = d0v[sl] * _lrelu(a1 + a2) + d1v[sl] * _lrelu(b1 + b2)
        efv[sl] = ef16
        mxv[...] = jnp.maximum(mxv[...], ef16)

    neg = jnp.full((L,), -3e38, jnp.float32)
    for r in range(8):
        for j in range(D // L):
            mxw[r, pl.ds(j * L, L)] = neg
    mxw[0, pl.ds(0, L)] = mxv[...]
    pltpu.sync_copy(mxw, mxsh.at[pl.ds(sid * 8, 8)])
    plsc.subcore_barrier()
    pltpu.sync_copy(mxsh, mxallv)
    m16 = mxallv[0, pl.ds(0, L)]
    for k in range(1, NS):
        m16 = jnp.maximum(m16, mxallv[k * 8, pl.ds(0, L)])
    mc = jnp.max(m16)

    @pl.loop(0, NIT)
    def _(it):
        eb = it * G

        @pl.loop(0, G // L)
        def _(g):
            sl = pl.ds(eb + g * L, L)
            lsl = pl.ds(g * L, L)
            ex16 = jnp.exp(efv[sl] - mc)
            d0v[sl] = ex16 * d0v[sl]
            d1v[sl] = ex16 * d1v[sl]
            d16 = dstv[sl]
            idxb[0, lsl] = lax.shift_right_logical(d16, 3)
            grp16 = lax.rem(d16, 8)
            for ri in range(L):
                r = g * L + ri
                exs = ex16[ri]
                grp = grp16[ri]
                for j in range(D // L):
                    denb[r, pl.ds(j * L, L)] = jnp.full(
                        (L,), jnp.where(grp == j, exs, 0.0), jnp.float32)

        pltpu.sync_copy(denb, sden.at[idxb.at[0]], add=True)

    pltpu.sync_copy(d0v, c0_hbm.at[pl.ds(base, EPW)])
    pltpu.sync_copy(d1v, c1_hbm.at[pl.ds(base, EPW)])
    pltpu.sync_copy(mxv, mx_hbm.at[wid])
    plsc.subcore_barrier()
    pltpu.sync_copy(sden.at[pl.ds(sid * RPS8, RPS8)],
                    pden_hbm.at[cid, pl.ds(sid * RPS8, RPS8)])


@functools.partial(
    pl.kernel,
    mesh=_mesh,
    out_type=[
        jax.ShapeDtypeStruct((NC, NPAD, D), jnp.float32),
    ],
    scratch_types=[
        pltpu.VMEM((4, G), jnp.int32),
        pltpu.VMEM((G, D), jnp.float32),
        pltpu.VMEM((G, D), jnp.float32),
        pltpu.VMEM_SHARED((NPAD, D), jnp.float32),
    ],
    compiler_params=_cp,
)
def _aggregate(z_hbm, zr_hbm, meta_hbm, pnum_hbm,
               mb0, za, zra, snum):
    cid = lax.axis_index("c")
    sid = lax.axis_index("s")
    wid = cid * NS + sid

    z16 = jnp.zeros((L,), jnp.float32)

    @pl.loop(0, G)
    def _(r):
        for j in range(D // L):
            za[r, pl.ds(j * L, L)] = z16

    for k in range(RPS // G):
        pltpu.sync_copy(za, snum.at[pl.ds(sid * RPS + k * G, G)])
    plsc.subcore_barrier()

    @pl.loop(0, NIT)
    def _(it):
        pltpu.sync_copy(meta_hbm.at[wid, it], mb0)
        pltpu.sync_copy(z_hbm.at[mb0.at[0]], za)
        pltpu.sync_copy(zr_hbm.at[mb0.at[0]], zra)

        @pl.loop(0, G // L)
        def _(g):
            lsl = pl.ds(g * L, L)
            c0_16 = plsc.bitcast(mb0[2, lsl], jnp.float32)
            c1_16 = plsc.bitcast(mb0[3, lsl], jnp.float32)
            for ri in range(L):
                r = g * L + ri
                c0 = c0_16[ri]
                c1 = c1_16[ri]
                for j in range(D // L):
                    sl2 = pl.ds(j * L, L)
                    za[r, sl2] = c0 * za[r, sl2] + c1 * zra[r, sl2]

        pltpu.sync_copy(za, snum.at[mb0.at[1]], add=True)

    plsc.subcore_barrier()
    pltpu.sync_copy(snum.at[pl.ds(sid * RPS, RPS)],
                    pnum_hbm.at[cid, pl.ds(sid * RPS, RPS)])


def _finalize_body(mx_ref, pn_ref, pd_ref, o_ref):
    mx = mx_ref[...]
    m0 = jnp.max(mx[:NS])
    m1 = jnp.max(mx[NS:])
    m = jnp.maximum(m0, m1)
    s0 = jnp.exp(m0 - m)
    s1 = jnp.exp(m1 - m)
    n = s0 * pn_ref[0] + s1 * pn_ref[1]
    d = s0 * pd_ref[0, :, 0:1] + s1 * pd_ref[1, :, 0:1]
    o_ref[...] = n / (d + 1e-38)


def _finalize(mx, pnum, pden):
    R = 1024
    return pl.pallas_call(
        _finalize_body,
        grid=(NPAD // R,),
        in_specs=[
            pl.BlockSpec((NW, L), lambda i: (0, 0)),
            pl.BlockSpec((NC, R, D), lambda i: (0, i, 0)),
            pl.BlockSpec((NC, R, L), lambda i: (0, i, 0)),
        ],
        out_specs=pl.BlockSpec((R, D), lambda i: (i, 0)),
        out_shape=jax.ShapeDtypeStruct((NPAD, D), jnp.float32),
    )(mx, pnum, pden)


def kernel(h, edge_index, direction, W_fc, W_fcr, W_attn):
    src = edge_index[0].astype(jnp.int32)
    dst = edge_index[1].astype(jnp.int32)
    d0 = direction[:, 0, 0]
    d1 = direction[:, 1, 0]
    am = W_attn.reshape(2, D).T

    z, zr, s = _project(h, W_fc, W_fcr, am)
    s1 = s[:, 0]
    s2 = s[:, 1]
    sr1 = s[:, 2]
    sr2 = s[:, 3]

    c0, c1, mx, pden = _edge_scores(s1, s2, sr1, sr2, src, dst, d0, d1)

    bc = lambda x: lax.bitcast_convert_type(x, jnp.int32)
    meta = jnp.stack([src, dst, bc(c0), bc(c1)], axis=0)
    meta = meta.reshape(4, NW, NIT, G).transpose(1, 2, 0, 3)
    pnum, = _aggregate(z, zr, meta)

    out = _finalize(mx, pnum, pden.reshape(NC, NPAD, L))
    return out[:N]
